# speedup vs baseline: 2.2880x; 2.2880x over previous
import jax
import jax.numpy as jnp
from jax.experimental import pallas as pl
from jax.experimental.pallas import tpu as pltpu

_EPS = 0.3 ** 2
_MAX_ITER = 100
_BLOCK_N = 8


def _sinkhorn_kernel(samples_ref, codebook_ref, out_ref):
    B, S, d = samples_ref.shape
    K = codebook_ref.shape[0]
    inv_eps = jnp.float32(1.0 / _EPS)
    log_a = -jnp.log(jnp.float32(S))
    log_b = -jnp.log(jnp.float32(K))

    x = samples_ref[...]
    y = codebook_ref[...]

    x2 = jnp.sum(x * x, axis=-1)
    y2 = jnp.sum(y * y, axis=-1)
    xy = jax.lax.dot_general(
        x.reshape(B * S, d), y,
        dimension_numbers=(((1,), (1,)), ((), ())),
        preferred_element_type=jnp.float32,
    )
    Ce = ((0.5 * x2.reshape(B * S, 1) - xy) + 0.5 * y2.reshape(1, K)) * inv_eps
    Ce = Ce.reshape(B, S, K)

    fe0 = jnp.zeros((B, S), jnp.float32)
    ge0 = jnp.zeros((B, K), jnp.float32)

    def body(_, carry):
        fe, ge = carry
        t = ge[:, None, :] - Ce
        m = jnp.max(t, axis=2)
        s = jnp.sum(jnp.exp(t - m[:, :, None]), axis=2)
        fe = -(m + jnp.log(s) + log_b)
        u = fe[:, :, None] - Ce
        m2 = jnp.max(u, axis=1)
        s2 = jnp.sum(jnp.exp(u - m2[:, None, :]), axis=1)
        ge = -(m2 + jnp.log(s2) + log_a)
        return fe, ge

    fe, ge = jax.lax.fori_loop(0, _MAX_ITER, body, (fe0, ge0))

    logP = fe[:, :, None] + ge[:, None, :] - Ce + (log_a + log_b)
    hist = jnp.sum(jnp.exp(logP), axis=1)
    hist = hist / jnp.sum(hist, axis=1, keepdims=True)
    out_ref[...] = hist


@jax.jit
def kernel(samples, codebook):
    N, S, d = samples.shape
    K = codebook.shape[0]
    grid = (N // _BLOCK_N,)
    return pl.pallas_call(
        _sinkhorn_kernel,
        grid=grid,
        in_specs=[
            pl.BlockSpec((_BLOCK_N, S, d), lambda i: (i, 0, 0)),
            pl.BlockSpec((K, d), lambda i: (0, 0)),
        ],
        out_specs=pl.BlockSpec((_BLOCK_N, K), lambda i: (i, 0)),
        out_shape=jax.ShapeDtypeStruct((N, K), jnp.float32),
        compiler_params=pltpu.CompilerParams(
            dimension_semantics=("parallel",),
        ),
    )(samples, codebook)


# speedup vs baseline: 2.4439x; 1.0681x over previous
import jax
import jax.numpy as jnp
from jax.experimental import pallas as pl
from jax.experimental.pallas import tpu as pltpu

_EPS = 0.3 ** 2
_MAX_ITER = 100
_BLOCK_N = 16


def _sinkhorn_kernel(samples_ref, codebook_ref, out_ref):
    B, S, d = samples_ref.shape
    K = codebook_ref.shape[0]
    log2e = jnp.float32(1.4426950408889634)
    scale = log2e / jnp.float32(_EPS)
    l2_a = -jnp.log2(jnp.float32(S))
    l2_b = -jnp.log2(jnp.float32(K))

    x = samples_ref[...]
    y = codebook_ref[...]

    x2 = jnp.sum(x * x, axis=-1)
    y2 = jnp.sum(y * y, axis=-1)
    xy = jax.lax.dot_general(
        x.reshape(B * S, d), y,
        dimension_numbers=(((1,), (1,)), ((), ())),
        preferred_element_type=jnp.float32,
    )
    Ce = ((0.5 * x2.reshape(B * S, 1) - xy) + 0.5 * y2.reshape(1, K)) * scale
    Ce = Ce.reshape(B, S, K)

    fe0 = jnp.zeros((B, S), jnp.float32)
    ge0 = jnp.zeros((B, K), jnp.float32)

    def body(_, carry):
        fe, ge = carry
        t = ge[:, None, :] - Ce
        m = jnp.max(t, axis=2)
        s = jnp.sum(jnp.exp2(t - m[:, :, None]), axis=2)
        fe = -(m + jnp.log2(s) + l2_b)
        u = fe[:, :, None] - Ce
        m2 = jnp.max(u, axis=1)
        s2 = jnp.sum(jnp.exp2(u - m2[:, None, :]), axis=1)
        ge = -(m2 + jnp.log2(s2) + l2_a)
        return fe, ge

    fe, ge = jax.lax.fori_loop(0, _MAX_ITER, body, (fe0, ge0))

    logP = fe[:, :, None] + ge[:, None, :] - Ce + (l2_a + l2_b)
    hist = jnp.sum(jnp.exp2(logP), axis=1)
    hist = hist / jnp.sum(hist, axis=1, keepdims=True)
    out_ref[...] = hist


@jax.jit
def kernel(samples, codebook):
    N, S, d = samples.shape
    K = codebook.shape[0]
    grid = (N // _BLOCK_N,)
    return pl.pallas_call(
        _sinkhorn_kernel,
        grid=grid,
        in_specs=[
            pl.BlockSpec((_BLOCK_N, S, d), lambda i: (i, 0, 0)),
            pl.BlockSpec((K, d), lambda i: (0, 0)),
        ],
        out_specs=pl.BlockSpec((_BLOCK_N, K), lambda i: (i, 0)),
        out_shape=jax.ShapeDtypeStruct((N, K), jnp.float32),
        compiler_params=pltpu.CompilerParams(
            dimension_semantics=("parallel",),
        ),
    )(samples, codebook)
